# baseline (speedup 1.0000x reference)
import functools

import jax
import jax.numpy as jnp
from jax.experimental import pallas as pl

NUM_ADJS = 3
N = 4096
IN_F = 512
OUT_F = 512

BM = 1024
BK = 512
KB = N // BK


def _support_kernel(x_ref, w_ref, s_ref):
    s_ref[0] = jnp.dot(
        x_ref[...], w_ref[0], preferred_element_type=jnp.float32
    ) * (1.0 / NUM_ADJS)


def _aggregate_kernel(a_ref, s_ref, b_ref, o_ref):
    r = pl.program_id(1)
    k = pl.program_id(2)
    prod = jnp.dot(a_ref[0], s_ref[0], preferred_element_type=jnp.float32)

    @pl.when((r == 0) & (k == 0))
    def _init():
        o_ref[...] = prod

    @pl.when(~((r == 0) & (k == 0)))
    def _acc():
        o_ref[...] += prod

    @pl.when((r == NUM_ADJS - 1) & (k == KB - 1))
    def _finish():
        o_ref[...] += b_ref[...]


@jax.jit
def kernel(input_, adjs, adj_weight, bias):
    support = pl.pallas_call(
        _support_kernel,
        grid=(NUM_ADJS,),
        in_specs=[
            pl.BlockSpec((N, IN_F), lambda i: (0, 0)),
            pl.BlockSpec((1, IN_F, OUT_F), lambda i: (i, 0, 0)),
        ],
        out_specs=pl.BlockSpec((1, N, OUT_F), lambda i: (i, 0, 0)),
        out_shape=jax.ShapeDtypeStruct((NUM_ADJS, N, OUT_F), jnp.float32),
    )(input_, adj_weight)

    bias2d = bias.reshape(1, OUT_F)

    out = pl.pallas_call(
        _aggregate_kernel,
        grid=(N // BM, NUM_ADJS, KB),
        in_specs=[
            pl.BlockSpec((1, BM, BK), lambda m, r, k: (r, m, k)),
            pl.BlockSpec((1, BK, OUT_F), lambda m, r, k: (r, k, 0)),
            pl.BlockSpec((1, OUT_F), lambda m, r, k: (0, 0)),
        ],
        out_specs=pl.BlockSpec((BM, OUT_F), lambda m, r, k: (m, 0)),
        out_shape=jax.ShapeDtypeStruct((N, OUT_F), jnp.float32),
    )(adjs, support, bias2d)

    return out


# speedup vs baseline: 1.0823x; 1.0823x over previous
import functools

import jax
import jax.numpy as jnp
from jax.experimental import pallas as pl

NUM_ADJS = 3
N = 4096
IN_F = 512
OUT_F = 512

BM = 1024
BK = 512
KB = N // BK


def _support_kernel(x_ref, w_ref, s_ref):
    prod = jnp.dot(
        x_ref[...].astype(jnp.bfloat16),
        w_ref[0].astype(jnp.bfloat16),
        preferred_element_type=jnp.float32,
    )
    s_ref[0] = (prod * (1.0 / NUM_ADJS)).astype(jnp.bfloat16)


def _aggregate_kernel(a_ref, s_ref, b_ref, o_ref):
    r = pl.program_id(1)
    k = pl.program_id(2)
    prod = jnp.dot(
        a_ref[0].astype(jnp.bfloat16),
        s_ref[0],
        preferred_element_type=jnp.float32,
    )

    @pl.when((r == 0) & (k == 0))
    def _init():
        o_ref[...] = prod

    @pl.when(~((r == 0) & (k == 0)))
    def _acc():
        o_ref[...] += prod

    @pl.when((r == NUM_ADJS - 1) & (k == KB - 1))
    def _finish():
        o_ref[...] += b_ref[...]


@jax.jit
def kernel(input_, adjs, adj_weight, bias):
    support = pl.pallas_call(
        _support_kernel,
        grid=(NUM_ADJS,),
        in_specs=[
            pl.BlockSpec((N, IN_F), lambda i: (0, 0)),
            pl.BlockSpec((1, IN_F, OUT_F), lambda i: (i, 0, 0)),
        ],
        out_specs=pl.BlockSpec((1, N, OUT_F), lambda i: (i, 0, 0)),
        out_shape=jax.ShapeDtypeStruct((NUM_ADJS, N, OUT_F), jnp.bfloat16),
    )(input_, adj_weight)

    bias2d = bias.reshape(1, OUT_F)

    out = pl.pallas_call(
        _aggregate_kernel,
        grid=(N // BM, NUM_ADJS, KB),
        in_specs=[
            pl.BlockSpec((1, BM, BK), lambda m, r, k: (r, m, k)),
            pl.BlockSpec((1, BK, OUT_F), lambda m, r, k: (r, k, 0)),
            pl.BlockSpec((1, OUT_F), lambda m, r, k: (0, 0)),
        ],
        out_specs=pl.BlockSpec((BM, OUT_F), lambda m, r, k: (m, 0)),
        out_shape=jax.ShapeDtypeStruct((N, OUT_F), jnp.float32),
    )(adjs, support, bias2d)

    return out


# speedup vs baseline: 1.8202x; 1.6818x over previous
import functools

import jax
import jax.numpy as jnp
from jax.experimental import pallas as pl

NUM_ADJS = 3
N = 4096
IN_F = 512
OUT_F = 512

BM = 256


def _support_kernel(x_ref, w_ref, s_ref):
    prod = jnp.dot(
        x_ref[...].astype(jnp.bfloat16),
        w_ref[0].astype(jnp.bfloat16),
        preferred_element_type=jnp.float32,
    )
    s_ref[0] = (prod * (1.0 / NUM_ADJS)).astype(jnp.bfloat16)


def _aggregate_kernel(a_ref, s_ref, b_ref, o_ref):
    acc = b_ref[...].astype(jnp.float32)
    for i in range(NUM_ADJS):
        acc = acc + jnp.dot(
            a_ref[i].astype(jnp.bfloat16),
            s_ref[i],
            preferred_element_type=jnp.float32,
        )
    o_ref[...] = acc


@jax.jit
def kernel(input_, adjs, adj_weight, bias):
    support = pl.pallas_call(
        _support_kernel,
        grid=(NUM_ADJS,),
        in_specs=[
            pl.BlockSpec((N, IN_F), lambda i: (0, 0)),
            pl.BlockSpec((1, IN_F, OUT_F), lambda i: (i, 0, 0)),
        ],
        out_specs=pl.BlockSpec((1, N, OUT_F), lambda i: (i, 0, 0)),
        out_shape=jax.ShapeDtypeStruct((NUM_ADJS, N, OUT_F), jnp.bfloat16),
    )(input_, adj_weight)

    bias2d = bias.reshape(1, OUT_F)

    out = pl.pallas_call(
        _aggregate_kernel,
        grid=(N // BM,),
        in_specs=[
            pl.BlockSpec((NUM_ADJS, BM, N), lambda m: (0, m, 0)),
            pl.BlockSpec((NUM_ADJS, N, OUT_F), lambda m: (0, 0, 0)),
            pl.BlockSpec((1, OUT_F), lambda m: (0, 0)),
        ],
        out_specs=pl.BlockSpec((BM, OUT_F), lambda m: (m, 0)),
        out_shape=jax.ShapeDtypeStruct((N, OUT_F), jnp.float32),
    )(adjs, support, bias2d)

    return out
